# baseline (speedup 1.0000x reference)
import functools

import jax
import jax.numpy as jnp
from jax import lax
from jax.experimental import pallas as pl
from jax.experimental.pallas import tpu as pltpu
from jax.experimental.pallas import tpu_sc as plsc

_NUM_EXPERTS = 64
_TOP_K = 2
_HIDDEN = 4096
_BT = 1024
_N_TOKENS = 32768
_NW = 32
_TPW = _N_TOKENS // 2 // _NW
_L = 16


def _matmul_body(x_ref, w_ref, lg_ref):
    lg_ref[...] = jax.lax.dot_general(
        x_ref[...], w_ref[...],
        dimension_numbers=(((1,), (1,)), ((), ())),
        preferred_element_type=jnp.float32,
        precision=jax.lax.Precision.DEFAULT,
    )


def _tc_logits(input, weight, n_tokens, block_offset):
    nb = n_tokens // _BT
    return pl.pallas_call(
        _matmul_body,
        grid=(nb,),
        in_specs=[
            pl.BlockSpec((_BT, _HIDDEN), lambda i: (i + block_offset, 0)),
            pl.BlockSpec((_NUM_EXPERTS, _HIDDEN), lambda i: (0, 0)),
        ],
        out_specs=pl.BlockSpec((_BT, _NUM_EXPERTS), lambda i: (i, 0)),
        out_shape=jax.ShapeDtypeStruct((n_tokens, _NUM_EXPERTS), jnp.float32),
        compiler_params=pltpu.CompilerParams(
            dimension_semantics=("arbitrary",),
            vmem_limit_bytes=60 * 1024 * 1024,
        ),
    )(input, weight)


@functools.partial(
    pl.kernel,
    mesh=plsc.VectorSubcoreMesh(core_axis_name="c", subcore_axis_name="s"),
    out_type=[
        jax.ShapeDtypeStruct((_N_TOKENS // 2 * _TOP_K,), jnp.float32),
        jax.ShapeDtypeStruct((_N_TOKENS // 2 * _TOP_K,), jnp.int32),
    ],
    scratch_types=[
        pltpu.VMEM((_TPW * _NUM_EXPERTS,), jnp.float32),
        pltpu.VMEM((_TPW * _TOP_K,), jnp.float32),
        pltpu.VMEM((_TPW * _TOP_K,), jnp.int32),
    ],
    compiler_params=pltpu.CompilerParams(needs_layout_passes=False),
)
def _sc_route_half(lg_hbm, tp_hbm, ti_hbm, slab, tp_buf, ti_buf):
    wid = lax.axis_index("s") * 2 + lax.axis_index("c")
    base = wid * _TPW
    pltpu.sync_copy(lg_hbm.at[pl.ds(base * _NUM_EXPERTS, _TPW * _NUM_EXPERTS)],
                    slab)

    lanes = lax.iota(jnp.int32, _L)
    neg_inf = jnp.full((_L,), -jnp.inf, jnp.float32)
    zero_i = jnp.zeros((_L,), jnp.int32)

    def group(g, carry):
        tok0 = (g * _L + lanes) * _NUM_EXPERTS
        rows = g * _L + lanes
        m1, i1, m2, i2 = neg_inf, zero_i, neg_inf, zero_i
        for e in range(_NUM_EXPERTS):
            v = plsc.load_gather(slab, [tok0 + e])
            gt1 = v > m1
            gt2 = v > m2
            e_vec = jnp.full((_L,), e, jnp.int32)
            i2 = jnp.where(gt1, i1, jnp.where(gt2, e_vec, i2))
            m2 = jnp.where(gt1, m1, jnp.where(gt2, v, m2))
            i1 = jnp.where(gt1, e_vec, i1)
            m1 = jnp.where(gt1, v, m1)
        z = jnp.zeros((_L,), jnp.float32)
        for e in range(_NUM_EXPERTS):
            v = plsc.load_gather(slab, [tok0 + e])
            z = z + jnp.exp(v - m1)
        p1 = jnp.exp(m1 - m1) / z
        p2 = jnp.exp(m2 - m1) / z
        out0 = (g * _L + lanes) * _TOP_K
        plsc.store_scatter(tp_buf, [out0], p1)
        plsc.store_scatter(tp_buf, [out0 + 1], p2)
        plsc.store_scatter(ti_buf, [out0], i1)
        plsc.store_scatter(ti_buf, [out0 + 1], i2)
        return carry

    lax.fori_loop(0, _TPW // _L, group, jnp.int32(0))

    pltpu.sync_copy(tp_buf, tp_hbm.at[pl.ds(base * _TOP_K, _TPW * _TOP_K)])
    pltpu.sync_copy(ti_buf, ti_hbm.at[pl.ds(base * _TOP_K, _TPW * _TOP_K)])


@jax.jit
def kernel(input, weight):
    half = _N_TOKENS // 2
    lg0 = _tc_logits(input, weight, half, 0)
    lg1 = _tc_logits(input, weight, half, half // _BT)
    tp0, ti0 = _sc_route_half(lg0.reshape(-1))
    tp1, ti1 = _sc_route_half(lg1.reshape(-1))
    lg = jnp.concatenate([lg0, lg1], axis=0)
    tp = jnp.concatenate([tp0, tp1], axis=0).reshape(half * 2, _TOP_K)
    ti = jnp.concatenate([ti0, ti1], axis=0).reshape(half * 2, _TOP_K)
    return tp, ti, lg


# speedup vs baseline: 1.3390x; 1.3390x over previous
import jax
import jax.numpy as jnp
from jax.experimental import pallas as pl
from jax.experimental.pallas import tpu as pltpu

_NUM_EXPERTS = 64
_TOP_K = 2
_HIDDEN = 4096
_BLOCK_T = 1024
_SUB = 32


def _router_body(x_ref, w_ref, tp_ref, ti_ref, lg_ref):
    x = x_ref[...]
    w = w_ref[...]
    logits = jax.lax.dot_general(
        x, w,
        dimension_numbers=(((1,), (1,)), ((), ())),
        preferred_element_type=jnp.float32,
        precision=jax.lax.Precision.DEFAULT,
    )
    lg_ref[...] = logits

    for s in range(0, _BLOCK_T, _SUB):
        l = logits[s:s + _SUB]
        m1 = jnp.max(l, axis=1, keepdims=True)
        z = jnp.sum(jnp.exp(l - m1), axis=1, keepdims=True)

        iota = jax.lax.broadcasted_iota(jnp.int32, l.shape, 1)
        sentinel = jnp.int32(_NUM_EXPERTS)
        i1 = jnp.min(jnp.where(l == m1, iota, sentinel), axis=1, keepdims=True)
        masked = jnp.where(iota == i1, -jnp.inf, l)
        m2 = jnp.max(masked, axis=1, keepdims=True)
        i2 = jnp.min(jnp.where(masked == m2, iota, sentinel), axis=1,
                     keepdims=True)

        p1 = jnp.exp(m1 - m1) / z
        p2 = jnp.exp(m2 - m1) / z
        tp_ref[s:s + _SUB, :] = jnp.concatenate([p1, p2], axis=1)
        ti_ref[s:s + _SUB, :] = jnp.concatenate([i1, i2], axis=1)


@jax.jit
def kernel(input, weight):
    n_tokens = input.shape[0]
    grid = (n_tokens // _BLOCK_T,)
    tp, ti, lg = pl.pallas_call(
        _router_body,
        grid=grid,
        in_specs=[
            pl.BlockSpec((_BLOCK_T, _HIDDEN), lambda i: (i, 0)),
            pl.BlockSpec((_NUM_EXPERTS, _HIDDEN), lambda i: (0, 0)),
        ],
        out_specs=[
            pl.BlockSpec((_BLOCK_T, _TOP_K), lambda i: (i, 0)),
            pl.BlockSpec((_BLOCK_T, _TOP_K), lambda i: (i, 0)),
            pl.BlockSpec((_BLOCK_T, _NUM_EXPERTS), lambda i: (i, 0)),
        ],
        out_shape=[
            jax.ShapeDtypeStruct((n_tokens, _TOP_K), jnp.float32),
            jax.ShapeDtypeStruct((n_tokens, _TOP_K), jnp.int32),
            jax.ShapeDtypeStruct((n_tokens, _NUM_EXPERTS), jnp.float32),
        ],
        compiler_params=pltpu.CompilerParams(
            dimension_semantics=("arbitrary",),
            vmem_limit_bytes=60 * 1024 * 1024,
        ),
    )(input, weight)
    return tp, ti, lg


# speedup vs baseline: 1.3398x; 1.0006x over previous
import jax
import jax.numpy as jnp
from jax.experimental import pallas as pl
from jax.experimental.pallas import tpu as pltpu

_NUM_EXPERTS = 64
_TOP_K = 2
_HIDDEN = 4096
_BLOCK_T = 1024
_SUB = 32


def _router_body(x_ref, w_ref, tp_ref, ti_ref, lg_ref):
    x = x_ref[...]
    w = w_ref[...]
    logits = jax.lax.dot_general(
        x, w,
        dimension_numbers=(((1,), (1,)), ((), ())),
        preferred_element_type=jnp.float32,
        precision=jax.lax.Precision.DEFAULT,
    )
    lg_ref[...] = logits

    for s in range(0, _BLOCK_T, _SUB):
        l = logits[s:s + _SUB]
        m1 = jnp.max(l, axis=1, keepdims=True)
        z = jnp.sum(jnp.exp(l - m1), axis=1, keepdims=True)

        iota = jax.lax.broadcasted_iota(jnp.int32, l.shape, 1)
        sentinel = jnp.int32(_NUM_EXPERTS)
        i1 = jnp.min(jnp.where(l == m1, iota, sentinel), axis=1, keepdims=True)
        masked = jnp.where(iota == i1, -jnp.inf, l)
        m2 = jnp.max(masked, axis=1, keepdims=True)
        i2 = jnp.min(jnp.where(masked == m2, iota, sentinel), axis=1,
                     keepdims=True)

        p1 = jnp.exp(m1 - m1) / z
        p2 = jnp.exp(m2 - m1) / z
        tp_ref[s:s + _SUB, :] = jnp.concatenate([p1, p2], axis=1)
        ti_ref[s:s + _SUB, :] = jnp.concatenate([i1, i2], axis=1)


@jax.jit
def kernel(input, weight):
    n_tokens = input.shape[0]
    grid = (n_tokens // _BLOCK_T,)
    tp, ti, lg = pl.pallas_call(
        _router_body,
        grid=grid,
        in_specs=[
            pl.BlockSpec((_BLOCK_T, _HIDDEN), lambda i: (i, 0)),
            pl.BlockSpec((_NUM_EXPERTS, _HIDDEN), lambda i: (0, 0)),
        ],
        out_specs=[
            pl.BlockSpec((_BLOCK_T, _TOP_K), lambda i: (i, 0)),
            pl.BlockSpec((_BLOCK_T, _TOP_K), lambda i: (i, 0)),
            pl.BlockSpec((_BLOCK_T, _NUM_EXPERTS), lambda i: (i, 0)),
        ],
        out_shape=[
            jax.ShapeDtypeStruct((n_tokens, _TOP_K), jnp.float32),
            jax.ShapeDtypeStruct((n_tokens, _TOP_K), jnp.int32),
            jax.ShapeDtypeStruct((n_tokens, _NUM_EXPERTS), jnp.float32),
        ],
        compiler_params=pltpu.CompilerParams(
            dimension_semantics=("parallel",),
            vmem_limit_bytes=60 * 1024 * 1024,
        ),
    )(input, weight)
    return tp, ti, lg


# speedup vs baseline: 1.3409x; 1.0008x over previous
import jax
import jax.numpy as jnp
from jax.experimental import pallas as pl
from jax.experimental.pallas import tpu as pltpu

_NUM_EXPERTS = 64
_TOP_K = 2
_HIDDEN = 4096
_BLOCK_T = 1024
_SUB = 32


def _router_body(x_ref, w_ref, tp_ref, ti_ref, lg_ref):
    x = x_ref[...]
    w = w_ref[...]
    logits = jax.lax.dot_general(
        x, w,
        dimension_numbers=(((1,), (1,)), ((), ())),
        preferred_element_type=jnp.float32,
        precision=jax.lax.Precision.DEFAULT,
    )
    lg_ref[...] = logits

    for s in range(0, _BLOCK_T, _SUB):
        l = logits[s:s + _SUB]
        m1 = jnp.max(l, axis=1, keepdims=True)
        z = jnp.sum(jnp.exp(l - m1), axis=1, keepdims=True)

        iota = jax.lax.broadcasted_iota(jnp.int32, l.shape, 1).astype(
            jnp.float32)
        sentinel = jnp.float32(_NUM_EXPERTS)
        i1 = jnp.min(jnp.where(l == m1, iota, sentinel), axis=1, keepdims=True)
        masked = jnp.where(iota == i1, -jnp.inf, l)
        m2 = jnp.max(masked, axis=1, keepdims=True)
        i2 = jnp.min(jnp.where(masked == m2, iota, sentinel), axis=1,
                     keepdims=True)

        p1 = 1.0 / z
        p2 = jnp.exp(m2 - m1) / z
        tp_ref[s:s + _SUB, :] = jnp.concatenate([p1, p2], axis=1)
        ti_ref[s:s + _SUB, :] = jnp.concatenate([i1, i2], axis=1).astype(
            jnp.int32)


@jax.jit
def kernel(input, weight):
    n_tokens = input.shape[0]
    grid = (n_tokens // _BLOCK_T,)
    tp, ti, lg = pl.pallas_call(
        _router_body,
        grid=grid,
        in_specs=[
            pl.BlockSpec((_BLOCK_T, _HIDDEN), lambda i: (i, 0)),
            pl.BlockSpec((_NUM_EXPERTS, _HIDDEN), lambda i: (0, 0)),
        ],
        out_specs=[
            pl.BlockSpec((_BLOCK_T, _TOP_K), lambda i: (i, 0)),
            pl.BlockSpec((_BLOCK_T, _TOP_K), lambda i: (i, 0)),
            pl.BlockSpec((_BLOCK_T, _NUM_EXPERTS), lambda i: (i, 0)),
        ],
        out_shape=[
            jax.ShapeDtypeStruct((n_tokens, _TOP_K), jnp.float32),
            jax.ShapeDtypeStruct((n_tokens, _TOP_K), jnp.int32),
            jax.ShapeDtypeStruct((n_tokens, _NUM_EXPERTS), jnp.float32),
        ],
        compiler_params=pltpu.CompilerParams(
            dimension_semantics=("arbitrary",),
            vmem_limit_bytes=60 * 1024 * 1024,
        ),
    )(input, weight)
    return tp, ti, lg
